# Initial kernel scaffold; baseline (speedup 1.0000x reference)
import functools
import math

import jax
import jax.numpy as jnp
import numpy as np
from jax import lax
from jax.experimental import pallas as pl
from jax.experimental.pallas import tpu as pltpu
from jax.experimental.pallas import tpu_sc as plsc

B = 4
L = 4096
D = 1024
N_ROWS = B * L
SCALE = math.sqrt(64.0)

_pos = np.arange(L, dtype=np.float32)[:, None]
_div = np.exp(
    np.arange(0, D, 2, dtype=np.float32) * (-math.log(10000.0) / D)
).astype(np.float32)
_PE = np.zeros((L, D), dtype=np.float32)
_PE[:, 0::2] = np.sin(_pos * _div)
_PE[:, 1::2] = np.cos(_pos * _div)

_info = plsc.get_sparse_core_info()
NC, NS, LANES = _info.num_cores, _info.num_subcores, _info.num_lanes
NW = NC * NS
L_PER_W = L // NW
CHUNK = 32
N_CHUNKS = L_PER_W // CHUNK
VREGS_PER_ROW = D // LANES


@functools.partial(
    pl.kernel,
    mesh=plsc.VectorSubcoreMesh(core_axis_name="c", subcore_axis_name="s"),
    out_type=jax.ShapeDtypeStruct((N_ROWS, D), jnp.float32),
    scratch_types=[
        pltpu.VMEM((CHUNK,), jnp.int32),
        pltpu.VMEM((CHUNK, D), jnp.float32),
        pltpu.VMEM((CHUNK, D), jnp.float32),
        pltpu.SemaphoreType.DMA,
    ],
)
def _emb_pe_kernel(src_hbm, table_hbm, pe_hbm, out_hbm, idx_v, pe_v, emb_v, sem):
    wid = lax.axis_index("s") * NC + lax.axis_index("c")
    for c in range(N_CHUNKS):
        lbase = wid * L_PER_W + c * CHUNK
        pltpu.sync_copy(pe_hbm.at[pl.ds(lbase, CHUNK)], pe_v)
        for b in range(B):
            row_base = b * L + lbase
            pltpu.sync_copy(src_hbm.at[pl.ds(row_base, CHUNK)], idx_v)
            pltpu.async_copy(table_hbm.at[idx_v], emb_v, sem).wait()

            def _row(r, _):
                for j in range(VREGS_PER_ROW):
                    sl = pl.ds(j * LANES, LANES)
                    emb_v[r, sl] = emb_v[r, sl] * SCALE + pe_v[r, sl]
                return _

            lax.fori_loop(0, CHUNK, _row, 0)
            pltpu.sync_copy(emb_v, out_hbm.at[pl.ds(row_base, CHUNK)])


def kernel(src, emb_table):
    src_flat = src.reshape(N_ROWS).astype(jnp.int32)
    pe = jnp.asarray(_PE)
    out = _emb_pe_kernel(src_flat, emb_table, pe)
    return out.reshape(B, L, D)


# speedup vs baseline: 1.5304x; 1.5304x over previous
import functools
import math

import jax
import jax.numpy as jnp
import numpy as np
from jax import lax
from jax.experimental import pallas as pl
from jax.experimental.pallas import tpu as pltpu
from jax.experimental.pallas import tpu_sc as plsc

B = 4
L = 4096
D = 1024
N_ROWS = B * L
SCALE = math.sqrt(64.0)

_pos = np.arange(L, dtype=np.float32)[:, None]
_div = np.exp(
    np.arange(0, D, 2, dtype=np.float32) * (-math.log(10000.0) / D)
).astype(np.float32)
_PE = np.zeros((L, D), dtype=np.float32)
_PE[:, 0::2] = np.sin(_pos * _div)
_PE[:, 1::2] = np.cos(_pos * _div)

NC, NS, LANES = 2, 16, 16
NW = NC * NS
L_PER_W = L // NW
CHUNK = 32
N_CHUNKS = L_PER_W // CHUNK
VREGS_PER_ROW = D // LANES


@functools.cache
def _build():
    @functools.partial(
        pl.kernel,
        mesh=plsc.VectorSubcoreMesh(core_axis_name="c", subcore_axis_name="s"),
        out_type=jax.ShapeDtypeStruct((N_ROWS, D), jnp.float32),
        scratch_types=[
            pltpu.VMEM((CHUNK,), jnp.int32),
            pltpu.VMEM((CHUNK, D), jnp.float32),
            pltpu.VMEM((CHUNK, D), jnp.float32),
            pltpu.SemaphoreType.DMA,
        ],
    )
    def _emb_pe_kernel(src_hbm, table_hbm, pe_hbm, out_hbm, idx_v, pe_v, emb_v, sem):
        wid = lax.axis_index("s") * NC + lax.axis_index("c")
        for c in range(N_CHUNKS):
            lbase = wid * L_PER_W + c * CHUNK
            pltpu.sync_copy(pe_hbm.at[pl.ds(lbase, CHUNK)], pe_v)
            for b in range(B):
                row_base = b * L + lbase
                pltpu.sync_copy(src_hbm.at[pl.ds(row_base, CHUNK)], idx_v)
                pltpu.async_copy(table_hbm.at[idx_v], emb_v, sem).wait()

                def _row(r, _):
                    for j in range(VREGS_PER_ROW):
                        sl = pl.ds(j * LANES, LANES)
                        emb_v[r, sl] = emb_v[r, sl] * SCALE + pe_v[r, sl]
                    return _

                lax.fori_loop(0, CHUNK, _row, 0)
                pltpu.sync_copy(emb_v, out_hbm.at[pl.ds(row_base, CHUNK)])

    return _emb_pe_kernel


def kernel(src, emb_table):
    src_flat = src.reshape(N_ROWS).astype(jnp.int32)
    pe = jnp.asarray(_PE)
    out = _build()(src_flat, emb_table, pe)
    return out.reshape(B, L, D)


# speedup vs baseline: 2.3814x; 1.5561x over previous
import functools
import math

import jax
import jax.numpy as jnp
import numpy as np
from jax import lax
from jax.experimental import pallas as pl
from jax.experimental.pallas import tpu as pltpu
from jax.experimental.pallas import tpu_sc as plsc

B = 4
L = 4096
D = 1024
N_ROWS = B * L
SCALE = math.sqrt(64.0)

_pos = np.arange(L, dtype=np.float32)[:, None]
_div = np.exp(
    np.arange(0, D, 2, dtype=np.float32) * (-math.log(10000.0) / D)
).astype(np.float32)
_PE = np.zeros((L, D), dtype=np.float32)
_PE[:, 0::2] = np.sin(_pos * _div)
_PE[:, 1::2] = np.cos(_pos * _div)

NC, NS, LANES = 2, 16, 16
NW = NC * NS
L_PER_W = L // NW
CHUNK = 16
N_CHUNKS = L_PER_W // CHUNK
VREGS_PER_ROW = D // LANES


@functools.cache
def _build():
    @functools.partial(
        pl.kernel,
        mesh=plsc.VectorSubcoreMesh(core_axis_name="c", subcore_axis_name="s"),
        out_type=jax.ShapeDtypeStruct((N_ROWS, D), jnp.float32),
        scratch_types=[
            pltpu.VMEM((B, L_PER_W), jnp.int32),
            pltpu.VMEM((CHUNK, D), jnp.float32),
            pltpu.VMEM((CHUNK, D), jnp.float32),
            pltpu.VMEM((CHUNK, D), jnp.float32),
            pltpu.VMEM((CHUNK, D), jnp.float32),
            pltpu.VMEM((CHUNK, D), jnp.float32),
            pltpu.VMEM((CHUNK, D), jnp.float32),
            pltpu.SemaphoreType.DMA,
            pltpu.SemaphoreType.DMA,
            pltpu.SemaphoreType.DMA,
            pltpu.SemaphoreType.DMA,
            pltpu.SemaphoreType.DMA,
            pltpu.SemaphoreType.DMA,
        ],
    )
    def _emb_pe_kernel(
        src_hbm, table_hbm, pe_hbm, out_hbm,
        idx_all, emb0, emb1, st0, st1, pe0, pe1,
        g0, g1, s0, s1, p0, p1,
    ):
        wid = lax.axis_index("s") * NC + lax.axis_index("c")
        base_l = wid * L_PER_W
        emb = (emb0, emb1)
        stg = (st0, st1)
        peb = (pe0, pe1)
        gs = (g0, g1)
        ss = (s0, s1)
        ps = (p0, p1)

        def idx_slice(c, b):
            return idx_all.at[b, pl.ds(c * CHUNK, CHUNK)]

        def issue_gather(c, b, p):
            pltpu.async_copy(table_hbm.at[idx_slice(c, b)], emb[p], gs[p])

        def wait_gather(c, b, p):
            pltpu.make_async_copy(
                table_hbm.at[idx_slice(c, b)], emb[p], gs[p]
            ).wait()

        def issue_pe(c, q):
            pltpu.async_copy(
                pe_hbm.at[pl.ds(base_l + c * CHUNK, CHUNK)], peb[q], ps[q]
            )

        def wait_pe(c, q):
            pltpu.make_async_copy(
                pe_hbm.at[pl.ds(base_l + c * CHUNK, CHUNK)], peb[q], ps[q]
            ).wait()

        def out_slice(c, b):
            return out_hbm.at[pl.ds(b * L + base_l + c * CHUNK, CHUNK)]

        def issue_store(c, b, p):
            pltpu.async_copy(stg[p], out_slice(c, b), ss[p])

        def wait_store(p):
            pltpu.make_async_copy(stg[p], out_slice(0, 0), ss[p]).wait()

        def emit_step(c, cc, b, first_block):
            p = b % 2
            if b < B - 1:
                issue_gather(c, b + 1, 1 - p)
            else:
                nc = (c + 1) % N_CHUNKS
                issue_gather(nc, 0, 0)
            if b == 0:
                issue_pe((c + 1) % N_CHUNKS, 1 - cc)
                wait_pe(c, cc)
            wait_gather(c, b, p)
            if not (first_block and cc == 0 and b < 2):
                wait_store(p)

            def _row(r, _):
                for j in range(VREGS_PER_ROW):
                    sl = pl.ds(j * LANES, LANES)
                    stg[p][r, sl] = emb[p][r, sl] * SCALE + peb[cc][r, sl]
                return _

            lax.fori_loop(0, CHUNK, _row, 0)
            issue_store(c, b, p)

        def emit_block(c2, first_block):
            for cc in range(2):
                c = 2 * c2 + cc
                for b in range(B):
                    emit_step(c, cc, b, first_block)

        for b in range(B):
            pltpu.sync_copy(
                src_hbm.at[pl.ds(b * L + base_l, L_PER_W)], idx_all.at[b]
            )
        issue_gather(0, 0, 0)
        issue_pe(0, 0)

        emit_block(0, True)

        def _c2_body(c2, _):
            emit_block(c2, False)
            return _

        lax.fori_loop(1, N_CHUNKS // 2, _c2_body, 0)

        wait_gather(0, 0, 0)
        wait_pe(0, 0)
        wait_store(0)
        wait_store(1)

    return _emb_pe_kernel


def kernel(src, emb_table):
    src_flat = src.reshape(N_ROWS).astype(jnp.int32)
    pe = jnp.asarray(_PE)
    out = _build()(src_flat, emb_table, pe)
    return out.reshape(B, L, D)


# speedup vs baseline: 2.4616x; 1.0337x over previous
import functools
import math

import jax
import jax.numpy as jnp
import numpy as np
from jax import lax
from jax.experimental import pallas as pl
from jax.experimental.pallas import tpu as pltpu
from jax.experimental.pallas import tpu_sc as plsc

B = 4
L = 4096
D = 1024
N_ROWS = B * L
SCALE = math.sqrt(64.0)

_pos = np.arange(L, dtype=np.float32)[:, None]
_div = np.exp(
    np.arange(0, D, 2, dtype=np.float32) * (-math.log(10000.0) / D)
).astype(np.float32)
_PE = np.zeros((L, D), dtype=np.float32)
_PE[:, 0::2] = np.sin(_pos * _div)
_PE[:, 1::2] = np.cos(_pos * _div)

NC, NS, LANES = 2, 16, 16
NW = NC * NS
L_PER_W = L // NW
CHUNK = 16
N_CHUNKS = L_PER_W // CHUNK
VREGS_PER_ROW = D // LANES
NBUF = 4


@functools.cache
def _build():
    @functools.partial(
        pl.kernel,
        mesh=plsc.VectorSubcoreMesh(core_axis_name="c", subcore_axis_name="s"),
        out_type=jax.ShapeDtypeStruct((N_ROWS, D), jnp.float32),
        scratch_types=[
            pltpu.VMEM((B, L_PER_W), jnp.int32),
            pltpu.VMEM((CHUNK, D), jnp.float32),
            pltpu.VMEM((CHUNK, D), jnp.float32),
            pltpu.VMEM((CHUNK, D), jnp.float32),
            pltpu.VMEM((CHUNK, D), jnp.float32),
            pltpu.VMEM((CHUNK, D), jnp.float32),
            pltpu.VMEM((CHUNK, D), jnp.float32),
            pltpu.SemaphoreType.DMA,
            pltpu.SemaphoreType.DMA,
            pltpu.SemaphoreType.DMA,
            pltpu.SemaphoreType.DMA,
            pltpu.SemaphoreType.DMA,
            pltpu.SemaphoreType.DMA,
            pltpu.SemaphoreType.DMA,
            pltpu.SemaphoreType.DMA,
            pltpu.SemaphoreType.DMA,
            pltpu.SemaphoreType.DMA,
        ],
    )
    def _emb_pe_kernel(
        src_hbm, table_hbm, pe_hbm, out_hbm,
        idx_all, r0, r1, r2, r3, pe0, pe1,
        g0, g1, g2, g3, s0, s1, s2, s3, p0, p1,
    ):
        wid = lax.axis_index("s") * NC + lax.axis_index("c")
        base_l = wid * L_PER_W
        row = (r0, r1, r2, r3)
        peb = (pe0, pe1)
        gs = (g0, g1, g2, g3)
        ss = (s0, s1, s2, s3)
        ps = (p0, p1)

        def idx_slice(c, b):
            return idx_all.at[b, pl.ds(c * CHUNK, CHUNK)]

        def issue_gather(c, b, p):
            pltpu.async_copy(table_hbm.at[idx_slice(c, b)], row[p], gs[p])

        def wait_gather(c, b, p):
            pltpu.make_async_copy(
                table_hbm.at[idx_slice(c, b)], row[p], gs[p]
            ).wait()

        def issue_pe(c, q):
            pltpu.async_copy(
                pe_hbm.at[pl.ds(base_l + c * CHUNK, CHUNK)], peb[q], ps[q]
            )

        def wait_pe(c, q):
            pltpu.make_async_copy(
                pe_hbm.at[pl.ds(base_l + c * CHUNK, CHUNK)], peb[q], ps[q]
            ).wait()

        def out_slice(c, b):
            return out_hbm.at[pl.ds(b * L + base_l + c * CHUNK, CHUNK)]

        def issue_store(c, b, p):
            pltpu.async_copy(row[p], out_slice(c, b), ss[p])

        def wait_store(p):
            pltpu.make_async_copy(row[p], out_slice(0, 0), ss[p]).wait()

        def emit_step(c, cc, b, first_block):
            pnext = (b + 2) % NBUF
            if not (first_block and cc == 0 and b < 2):
                wait_store(pnext)
            if b < 2:
                issue_gather(c, b + 2, pnext)
            else:
                issue_gather((c + 1) % N_CHUNKS, b - 2, pnext)
            if b == 0:
                issue_pe((c + 1) % N_CHUNKS, 1 - cc)
                wait_pe(c, cc)
            wait_gather(c, b, b)

            def _row(r, _):
                for j in range(VREGS_PER_ROW):
                    sl = pl.ds(j * LANES, LANES)
                    row[b][r, sl] = row[b][r, sl] * SCALE + peb[cc][r, sl]
                return _

            lax.fori_loop(0, CHUNK, _row, 0)
            issue_store(c, b, b)

        def emit_block(c2, first_block):
            for cc in range(2):
                c = 2 * c2 + cc
                for b in range(B):
                    emit_step(c, cc, b, first_block)

        for b in range(B):
            pltpu.sync_copy(
                src_hbm.at[pl.ds(b * L + base_l, L_PER_W)], idx_all.at[b]
            )
        issue_gather(0, 0, 0)
        issue_gather(0, 1, 1)
        issue_pe(0, 0)

        emit_block(0, True)

        def _c2_body(c2, _):
            emit_block(c2, False)
            return _

        lax.fori_loop(1, N_CHUNKS // 2, _c2_body, 0)

        wait_gather(0, 0, 0)
        wait_gather(0, 1, 1)
        wait_pe(0, 0)
        wait_store(2)
        wait_store(3)

    return _emb_pe_kernel


def kernel(src, emb_table):
    src_flat = src.reshape(N_ROWS).astype(jnp.int32)
    pe = jnp.asarray(_PE)
    out = _build()(src_flat, emb_table, pe)
    return out.reshape(B, L, D)


# speedup vs baseline: 2.5183x; 1.0230x over previous
import functools
import math

import jax
import jax.numpy as jnp
import numpy as np
from jax import lax
from jax.experimental import pallas as pl
from jax.experimental.pallas import tpu as pltpu
from jax.experimental.pallas import tpu_sc as plsc

B = 4
L = 4096
D = 1024
N_ROWS = B * L
SCALE = math.sqrt(64.0)

_pos = np.arange(L, dtype=np.float32)[:, None]
_div = np.exp(
    np.arange(0, D, 2, dtype=np.float32) * (-math.log(10000.0) / D)
).astype(np.float32)
_PE = np.zeros((L, D), dtype=np.float32)
_PE[:, 0::2] = np.sin(_pos * _div)
_PE[:, 1::2] = np.cos(_pos * _div)

NC, NS, LANES = 2, 16, 16
NW = NC * NS
L_PER_W = L // NW
P = 4
RPS = B * P
N_STEPS = L_PER_W // P
VREGS_PER_ROW = D // LANES
NBUF = 4


@functools.cache
def _build():
    @functools.partial(
        pl.kernel,
        mesh=plsc.VectorSubcoreMesh(core_axis_name="c", subcore_axis_name="s"),
        out_type=jax.ShapeDtypeStruct((N_ROWS, D), jnp.float32),
        scratch_types=[
            pltpu.VMEM((N_STEPS * RPS,), jnp.int32),
            pltpu.VMEM((RPS, D), jnp.float32),
            pltpu.VMEM((RPS, D), jnp.float32),
            pltpu.VMEM((RPS, D), jnp.float32),
            pltpu.VMEM((RPS, D), jnp.float32),
            pltpu.VMEM((P, D), jnp.float32),
            pltpu.VMEM((P, D), jnp.float32),
            pltpu.SemaphoreType.DMA,
            pltpu.SemaphoreType.DMA,
            pltpu.SemaphoreType.DMA,
            pltpu.SemaphoreType.DMA,
            pltpu.SemaphoreType.DMA,
            pltpu.SemaphoreType.DMA,
            pltpu.SemaphoreType.DMA,
            pltpu.SemaphoreType.DMA,
            pltpu.SemaphoreType.DMA,
            pltpu.SemaphoreType.DMA,
        ],
    )
    def _emb_pe_kernel(
        src_hbm, table_hbm, pe_hbm, out_hbm,
        idx_all, r0, r1, r2, r3, pe0, pe1,
        g0, g1, g2, g3, s0, s1, s2, s3, p0, p1,
    ):
        wid = lax.axis_index("s") * NC + lax.axis_index("c")
        base_l = wid * L_PER_W
        row = (r0, r1, r2, r3)
        peb = (pe0, pe1)
        gs = (g0, g1, g2, g3)
        ss = (s0, s1, s2, s3)
        ps = (p0, p1)

        def idx_slice(c):
            return idx_all.at[pl.ds(c * RPS, RPS)]

        def issue_gather(c, q):
            pltpu.async_copy(table_hbm.at[idx_slice(c)], row[q], gs[q])

        def wait_gather(c, q):
            pltpu.make_async_copy(
                table_hbm.at[idx_slice(c)], row[q], gs[q]
            ).wait()

        def issue_pe(c, q):
            pltpu.async_copy(
                pe_hbm.at[pl.ds(base_l + c * P, P)], peb[q], ps[q]
            )

        def wait_pe(c, q):
            pltpu.make_async_copy(
                pe_hbm.at[pl.ds(base_l + c * P, P)], peb[q], ps[q]
            ).wait()

        def issue_stores(c, q):
            for b in range(B):
                pltpu.async_copy(
                    row[q].at[pl.ds(b * P, P)],
                    out_hbm.at[pl.ds(b * L + base_l + c * P, P)],
                    ss[q],
                )

        def wait_stores(q):
            for _ in range(B):
                pltpu.make_async_copy(
                    row[q].at[pl.ds(0, P)], out_hbm.at[pl.ds(0, P)], ss[q]
                ).wait()

        def emit_step(c, q, qpe, first):
            qn = (q + 2) % NBUF
            if not (first and q < 2):
                wait_stores(qn)
            issue_gather((c + 2) % N_STEPS, qn)
            issue_pe((c + 1) % N_STEPS, 1 - qpe)
            wait_pe(c, qpe)
            wait_gather(c, q)

            def _col(j, _):
                sl = pl.ds(j * LANES, LANES)
                for p in range(P):
                    pe_v = peb[qpe][p, sl]
                    for b in range(B):
                        r = b * P + p
                        row[q][r, sl] = row[q][r, sl] * SCALE + pe_v
                return _

            lax.fori_loop(0, VREGS_PER_ROW, _col, 0)
            issue_stores(c, q)

        def emit_group(c4, first):
            for u in range(NBUF):
                emit_step(c4 * NBUF + u, u, u % 2, first)

        pltpu.sync_copy(
            src_hbm.at[pl.ds(wid * (N_STEPS * RPS), N_STEPS * RPS)], idx_all
        )
        issue_gather(0, 0)
        issue_gather(1, 1)
        issue_pe(0, 0)

        emit_group(0, True)

        def _c4_body(c4, _):
            emit_group(c4, False)
            return _

        lax.fori_loop(1, N_STEPS // NBUF, _c4_body, 0)

        wait_gather(0, 0)
        wait_gather(1, 1)
        wait_pe(0, 0)
        wait_stores(2)
        wait_stores(3)

    return _emb_pe_kernel


def kernel(src, emb_table):
    src_arr = (
        src.reshape(B, NW, N_STEPS, P)
        .transpose(1, 2, 0, 3)
        .reshape(N_ROWS)
        .astype(jnp.int32)
    )
    pe = jnp.asarray(_PE)
    out = _build()(src_arr, emb_table, pe)
    return out.reshape(B, L, D)
